# baseline (speedup 1.0000x reference)
import functools

import jax
import jax.numpy as jnp
from jax import lax
from jax.experimental import pallas as pl
from jax.experimental.pallas import tpu as pltpu
from jax.experimental.pallas import tpu_sc as plsc

N = 10000
G = 64
EPS = 1e-5
NP = 10240
NC, NS = 2, 16
NW = NC * NS
CH = 112
BN = 512
ROWS_PER_S = NP // NS
WBR = 80
WB_CH = ROWS_PER_S // WBR


def _sc_pass(D, with_cnt, KJ, kj_by_core, bf16=False, nh=1):
    mesh = plsc.VectorSubcoreMesh(
        core_axis_name="c", subcore_axis_name="s",
        num_cores=NC, num_subcores=NS)

    gw = D // 2 if bf16 else D
    scratch = [
        pltpu.VMEM((4, CH), jnp.int32),
        pltpu.VMEM((4, CH), jnp.int32),
        pltpu.VMEM((CH,), jnp.int32),
        pltpu.VMEM((CH,), jnp.int32),
        pltpu.VMEM((CH, gw), jnp.int32),
        pltpu.VMEM((CH, gw), jnp.int32),
        pltpu.VMEM((CH, D), jnp.float32),
        pltpu.VMEM((CH, D), jnp.float32),
        pltpu.VMEM_SHARED((NP, D), jnp.float32),
        pltpu.SemaphoreType.DMA,
        pltpu.SemaphoreType.DMA,
        pltpu.SemaphoreType.DMA,
        pltpu.SemaphoreType.DMA,
        pltpu.SemaphoreType.DMA,
        pltpu.SemaphoreType.DMA,
        pltpu.SemaphoreType.DMA,
        pltpu.SemaphoreType.DMA,
    ]

    def body(*args):
        h_hbms = args[:nh]
        (idx_hbm, out_hbm, ri0, ri1, sr0, sr1, bb0, bb1,
         buf0, buf1, acc, is0, is1, gs0a, gs0b, gs1a, gs1b,
         ss0, ss1) = args[nh:]
        gs0 = (gs0a, gs0b)
        gs1 = (gs1a, gs1b)
        c = lax.axis_index("c")
        s = lax.axis_index("s")
        w = s * NC + c

        zero16 = jnp.zeros((16,), jnp.float32)

        @pl.loop(0, CH)
        def _zero_buf(r):
            for k in range(D // 16):
                buf0[r, pl.ds(k * 16, 16)] = zero16

        lane = lax.iota(jnp.int32, 16)
        is_lane3 = lane == 3
        two_v = jnp.full((16,), 2, jnp.int32)
        three_v = jnp.full((16,), 3, jnp.int32)

        def run_half(h_hbm, ohalf):
            @pl.loop(0, WB_CH)
            def _zero_acc(t):
                pltpu.sync_copy(
                    buf0.at[pl.ds(0, WBR)],
                    acc.at[pl.ds(s * ROWS_PER_S + t * WBR, WBR)])

            plsc.subcore_barrier()
            _run_edges(h_hbm)
            plsc.subcore_barrier()

            @pl.loop(0, WB_CH)
            def _writeback(t):
                base = s * ROWS_PER_S + t * WBR
                pltpu.sync_copy(acc.at[pl.ds(base, WBR)],
                                buf0.at[pl.ds(0, WBR)])
                pltpu.sync_copy(buf0.at[pl.ds(0, WBR)],
                                ohalf.at[c, pl.ds(base, WBR)])

        def istart(j, ri, sem):
            pltpu.async_copy(idx_hbm.at[w, j], ri, sem)

        def iwait(ri, sem):
            pltpu.make_async_copy(idx_hbm.at[w, 0], ri, sem).wait()

        gh = CH // 2

        def gstart(h_hbm, ri, bb, sem):
            pltpu.async_copy(h_hbm.at[ri.at[0, pl.ds(0, gh)]],
                             bb.at[pl.ds(0, gh)], sem[0])
            pltpu.async_copy(h_hbm.at[ri.at[0, pl.ds(gh, gh)]],
                             bb.at[pl.ds(gh, gh)], sem[1])

        def gwait(h_hbm, bb, sem):
            pltpu.make_async_copy(h_hbm.at[ri0.at[0, pl.ds(0, gh)]],
                                  bb.at[pl.ds(0, gh)], sem[0]).wait()
            pltpu.make_async_copy(h_hbm.at[ri0.at[0, pl.ds(gh, gh)]],
                                  bb.at[pl.ds(gh, gh)], sem[1]).wait()

        def sstart(buf, sr, sem):
            pltpu.async_copy(buf, acc.at[sr], sem, add=True)

        def swait(buf, sem):
            pltpu.make_async_copy(buf, acc.at[sr0], sem).wait()

        hmask = jnp.int32(-65536)

        def scale(ri, bb, buf):
            @plsc.parallel_loop(0, CH, unroll=16)
            def _edge(e):
                ev = jnp.full((16,), e, jnp.int32)
                ews = plsc.bitcast(
                    plsc.load_gather(ri, [two_v, ev]), jnp.float32)
                if with_cnt:
                    vals = plsc.bitcast(
                        plsc.load_gather(ri, [three_v, ev]), jnp.float32)
                    row = plsc.bitcast(bb[e, pl.ds(0, 16)], jnp.float32)
                    buf[e, pl.ds(0, 16)] = jnp.where(
                        is_lane3, vals, row * ews)
                elif bf16:
                    for k in range(D // 32):
                        u = bb[e, pl.ds(k * 16, 16)]
                        lo = plsc.bitcast(lax.shift_left(u, 16),
                                          jnp.float32)
                        hi = plsc.bitcast(u & hmask, jnp.float32)
                        buf[e, pl.ds(k * 32, 16)] = lo * ews
                        buf[e, pl.ds(k * 32 + 16, 16)] = hi * ews
                else:
                    for k in range(D // 16):
                        buf[e, pl.ds(k * 16, 16)] = plsc.bitcast(
                            bb[e, pl.ds(k * 16, 16)], jnp.float32) * ews

        def dstcopy(ri, sr):
            for k in range(CH // 16):
                sr[pl.ds(k * 16, 16)] = ri[1, pl.ds(k * 16, 16)]

        def _run_edges(h_hbm):
            kjc = lax.select(c == 0, jnp.int32(kj_by_core[0]),
                             jnp.int32(kj_by_core[1]))
            pltpu.sync_copy(idx_hbm.at[w, 0], ri0)
            gstart(h_hbm, ri0, bb0, gs0)
            istart(1, ri1, is1)

            @pl.loop(0, kjc, step=2)
            def _pair(j):
                iwait(ri1, is1)

                @pl.when(j > 0)
                def _():
                    swait(buf1, ss1)
                gstart(h_hbm, ri1, bb1, gs1)
                gwait(h_hbm, bb0, gs0)

                @pl.when(j > 1)
                def _():
                    swait(buf0, ss0)
                scale(ri0, bb0, buf0)
                dstcopy(ri0, sr0)
                sstart(buf0, sr0, ss0)

                @pl.when(j + 2 < kjc)
                def _():
                    istart(j + 2, ri0, is0)

                gwait(h_hbm, bb1, gs1)
                scale(ri1, bb1, buf1)
                dstcopy(ri1, sr1)
                sstart(buf1, sr1, ss1)

                @pl.when(j + 2 < kjc)
                def _():
                    iwait(ri0, is0)
                    gstart(h_hbm, ri0, bb0, gs0)

                @pl.when(j + 3 < kjc)
                def _():
                    istart(j + 3, ri1, is1)

            swait(buf0, ss0)
            swait(buf1, ss1)

        if nh == 1:
            run_half(h_hbms[0], out_hbm)
        else:
            for t in range(nh):
                run_half(h_hbms[t], out_hbm.at[t])

    oshape = ((NC, NP, D) if nh == 1 else (nh, NC, NP, D))
    return pl.kernel(
        body,
        out_type=jax.ShapeDtypeStruct(oshape, jnp.float32),
        mesh=mesh,
        scratch_types=scratch,
        compiler_params=pltpu.CompilerParams(
            needs_layout_passes=False, use_tc_tiling_on_sc=False),
    )


def _layer1_call(p, x_pad, wr, wt, bias):
    def body(p_ref, x_ref, wr_ref, wt_ref, b_ref, out_ref, inv_ref):
        tot = p_ref[0] + p_ref[1]
        cnt = tot[:, 3:4]
        inv = 1.0 / jnp.maximum(cnt, 1.0)
        mean = tot * inv
        z = (jnp.dot(mean.astype(jnp.bfloat16), wr_ref[...],
                     preferred_element_type=jnp.float32)
             + jnp.dot(x_ref[...].astype(jnp.bfloat16), wt_ref[...],
                       preferred_element_type=jnp.float32)
             + b_ref[...])
        out_ref[...] = jnp.maximum(z, 0.0)
        inv_ref[...] = inv

    grid = NP // BN
    return pl.pallas_call(
        body,
        grid=(grid,),
        in_specs=[
            pl.BlockSpec((NC, BN, 16), lambda i: (0, i, 0)),
            pl.BlockSpec((BN, 16), lambda i: (i, 0)),
            pl.BlockSpec((16, 32), lambda i: (0, 0)),
            pl.BlockSpec((16, 32), lambda i: (0, 0)),
            pl.BlockSpec((1, 32), lambda i: (0, 0)),
        ],
        out_specs=[
            pl.BlockSpec((BN, 32), lambda i: (i, 0)),
            pl.BlockSpec((BN, 1), lambda i: (i, 0)),
        ],
        out_shape=[
            jax.ShapeDtypeStruct((NP, 32), jnp.float32),
            jax.ShapeDtypeStruct((NP, 1), jnp.float32),
        ],
    )(p, x_pad, wr, wt, bias)


def _layer_call(parts, hs, invc, wrs, wts, bias, dout, n_split):
    n_p, n_h = len(parts), len(hs)
    dps = [a.shape[2] for a in parts]
    dhs = [a.shape[1] for a in hs]
    wsp = dout // n_split

    def body(*refs):
        p_refs = refs[:n_p]
        h_refs = refs[n_p:n_p + n_h]
        inv_ref = refs[n_p + n_h]
        wr_refs = refs[n_p + n_h + 1: n_p + n_h + 1 + n_p]
        wt_refs = refs[n_p + n_h + 1 + n_p: n_p + n_h + 1 + n_p + n_h]
        b_ref = refs[n_p + n_h + 1 + n_p + n_h]
        out_refs = refs[n_p + n_h + 2 + n_p + n_h:]

        inv = inv_ref[...]
        z = b_ref[...]
        acc = None
        for pr, wr in zip(p_refs, wr_refs):
            mean = ((pr[0] + pr[1]) * inv).astype(jnp.bfloat16)
            t = jnp.dot(mean, wr[...], preferred_element_type=jnp.float32)
            acc = t if acc is None else acc + t
        for hr, wt in zip(h_refs, wt_refs):
            acc = acc + jnp.dot(hr[...].astype(jnp.bfloat16), wt[...],
                                preferred_element_type=jnp.float32)
        out = jnp.maximum(acc + z, 0.0)
        for k, o in enumerate(out_refs):
            o[...] = out[:, k * wsp:(k + 1) * wsp]

    grid = NP // BN
    in_specs = (
        [pl.BlockSpec((NC, BN, d), lambda i: (0, i, 0)) for d in dps]
        + [pl.BlockSpec((BN, d), lambda i: (i, 0)) for d in dhs]
        + [pl.BlockSpec((BN, 1), lambda i: (i, 0))]
        + [pl.BlockSpec((d, dout), lambda i: (0, 0)) for d in dps]
        + [pl.BlockSpec((d, dout), lambda i: (0, 0)) for d in dhs]
        + [pl.BlockSpec((1, dout), lambda i: (0, 0))]
    )
    out_specs = [pl.BlockSpec((BN, wsp), lambda i: (i, 0))
                 for _ in range(n_split)]
    out_shape = [jax.ShapeDtypeStruct((NP, wsp), jnp.float32)
                 for _ in range(n_split)]
    res = pl.pallas_call(
        body,
        grid=(grid,),
        in_specs=in_specs,
        out_specs=out_specs,
        out_shape=out_shape,
    )(*parts, *hs, invc, *wrs, *wts, bias)
    return res


def _layer5_pool_call(parts, hs, invc, wrs, wts, bias, batch3):
    nblk = NP // BN

    def body(pa, pb, ha, hb, inv_ref, wra, wrb, wta, wtb, b_ref, b3_ref,
             sum_ref, cnt_ref):
        i = pl.program_id(0)
        inv = inv_ref[...]
        acc = jnp.dot(((pa[0] + pa[1]) * inv).astype(jnp.bfloat16),
                      wra[...], preferred_element_type=jnp.float32)
        acc += jnp.dot(((pb[0] + pb[1]) * inv).astype(jnp.bfloat16),
                       wrb[...], preferred_element_type=jnp.float32)
        acc += jnp.dot(ha[...].astype(jnp.bfloat16), wta[...],
                       preferred_element_type=jnp.float32)
        acc += jnp.dot(hb[...].astype(jnp.bfloat16), wtb[...],
                       preferred_element_type=jnp.float32)
        out = jnp.maximum(acc + b_ref[...], 0.0)

        bv = b3_ref[0, 0, :]
        oh = (bv[:, None]
              == lax.broadcasted_iota(jnp.int32, (BN, G), 1)
              ).astype(jnp.float32)
        ps = lax.dot_general(oh, out, (((0,), (0,)), ((), ())),
                             preferred_element_type=jnp.float32)
        pc = jnp.sum(oh, axis=0)[:, None]

        @pl.when(i == 0)
        def _():
            sum_ref[...] = ps
            cnt_ref[...] = pc

        @pl.when(i != 0)
        def _():
            sum_ref[...] += ps
            cnt_ref[...] += pc

        @pl.when(i == nblk - 1)
        def _():
            sum_ref[...] = sum_ref[...] / jnp.maximum(cnt_ref[...], 1.0)

    return pl.pallas_call(
        body,
        grid=(nblk,),
        in_specs=[
            pl.BlockSpec((NC, BN, 128), lambda i: (0, i, 0)),
            pl.BlockSpec((NC, BN, 128), lambda i: (0, i, 0)),
            pl.BlockSpec((BN, 128), lambda i: (i, 0)),
            pl.BlockSpec((BN, 128), lambda i: (i, 0)),
            pl.BlockSpec((BN, 1), lambda i: (i, 0)),
            pl.BlockSpec((128, 512), lambda i: (0, 0)),
            pl.BlockSpec((128, 512), lambda i: (0, 0)),
            pl.BlockSpec((128, 512), lambda i: (0, 0)),
            pl.BlockSpec((128, 512), lambda i: (0, 0)),
            pl.BlockSpec((1, 512), lambda i: (0, 0)),
            pl.BlockSpec((1, 1, BN), lambda i: (i, 0, 0)),
        ],
        out_specs=[
            pl.BlockSpec((G, 512), lambda i: (0, 0)),
            pl.BlockSpec((G, 1), lambda i: (0, 0)),
        ],
        out_shape=[
            jax.ShapeDtypeStruct((G, 512), jnp.float32),
            jax.ShapeDtypeStruct((G, 1), jnp.float32),
        ],
    )(*parts, *hs, invc, *wrs, *wts, bias, batch3)[0]


def _pool_call(h5, batch3):
    nblk = NP // BN

    def body(h_ref, b_ref, sum_ref, cnt_ref):
        i = pl.program_id(0)
        bv = b_ref[0, 0, :]
        oh = (bv[:, None]
              == lax.broadcasted_iota(jnp.int32, (BN, G), 1)
              ).astype(jnp.float32)
        ps = lax.dot_general(oh, h_ref[...],
                             (((0,), (0,)), ((), ())),
                             preferred_element_type=jnp.float32)
        pc = jnp.sum(oh, axis=0)[:, None]

        @pl.when(i == 0)
        def _():
            sum_ref[...] = ps
            cnt_ref[...] = pc

        @pl.when(i != 0)
        def _():
            sum_ref[...] += ps
            cnt_ref[...] += pc

        @pl.when(i == nblk - 1)
        def _():
            sum_ref[...] = sum_ref[...] / jnp.maximum(cnt_ref[...], 1.0)

    return pl.pallas_call(
        body,
        grid=(nblk,),
        in_specs=[
            pl.BlockSpec((BN, 512), lambda i: (i, 0)),
            pl.BlockSpec((1, 1, BN), lambda i: (i, 0, 0)),
        ],
        out_specs=[
            pl.BlockSpec((G, 512), lambda i: (0, 0)),
            pl.BlockSpec((G, 1), lambda i: (0, 0)),
        ],
        out_shape=[
            jax.ShapeDtypeStruct((G, 512), jnp.float32),
            jax.ShapeDtypeStruct((G, 1), jnp.float32),
        ],
    )(h5, batch3)[0]


def _mlp_call(pooled, ws, bs):
    def body(x_ref, w1, b1, w2, b2, w3, b3, w4, b4, out_ref):
        h = x_ref[...]
        for wref, bref in ((w1, b1), (w2, b2), (w3, b3)):
            h = jnp.maximum(
                jnp.dot(h, wref[...], preferred_element_type=jnp.float32)
                + bref[...], 0.0)
        z = (jnp.dot(h, w4[...], preferred_element_type=jnp.float32)
             + b4[...])
        m = jnp.max(z, axis=1, keepdims=True)
        zz = z - m
        out_ref[...] = zz - jnp.log(
            jnp.sum(jnp.exp(zz), axis=1, keepdims=True))

    args = [pooled]
    for w, b in zip(ws, bs):
        args += [w, b]
    return pl.pallas_call(
        body,
        out_shape=jax.ShapeDtypeStruct((G, 2), jnp.float32),
    )(*args)


def _pack_bf16(h):
    npad, d = h.shape
    hb = h.reshape(npad, d // 32, 2, 16).swapaxes(-2, -1).astype(
        jnp.bfloat16)
    return lax.bitcast_convert_type(hb, jnp.int32).reshape(npad, d // 2)


def _fold_bn(w, b_lin, g, b_bn):
    s = g / jnp.sqrt(1.0 + EPS)
    wf = (w * s[:, None]).T
    bf = (b_lin * s + b_bn)[None, :]
    return wf, bf


def _slabify(a, kj0, kj1, kjm):
    a0 = a[:NS * kj0 * CH].reshape(NS, kj0, CH)
    a1 = a[NS * kj0 * CH:].reshape(NS, kj1, CH)
    a0 = jnp.pad(a0, ((0, 0), (0, kjm - kj0), (0, 0)))
    a1 = jnp.pad(a1, ((0, 0), (0, kjm - kj1), (0, 0)))
    return jnp.stack([a0, a1], axis=1).reshape(NW, kjm, CH)


def kernel(x, edge_index, edge_weight, edge_attr, batch, params):
    del edge_attr
    E = edge_index.shape[1]
    tch = 2 * ((E + 2 * NS * CH - 1) // (2 * NS * CH))
    kj0 = 2 * int(round(0.35 * tch))
    kj1 = tch - kj0
    kjm = max(kj0, kj1)
    e_cap = NS * CH * tch
    pe = e_cap - E

    src_f = jnp.pad(edge_index[0], (0, pe)).astype(jnp.int32)
    pad_dst = N + (jnp.arange(pe, dtype=jnp.int32) % (NP - N))
    dst_f = jnp.concatenate([edge_index[1].astype(jnp.int32), pad_dst])
    ew_f = jnp.pad(edge_weight, (0, pe))
    val_f = jnp.pad(jnp.ones((E,), jnp.float32), (0, pe))

    idx4 = jnp.stack(
        [_slabify(src_f, kj0, kj1, kjm),
         _slabify(dst_f, kj0, kj1, kjm),
         _slabify(lax.bitcast_convert_type(ew_f, jnp.int32),
                  kj0, kj1, kjm),
         _slabify(lax.bitcast_convert_type(val_f, jnp.int32),
                  kj0, kj1, kjm)], axis=2)
    kj = kjm
    kjbc = (kj0, kj1)

    x_pad = jnp.pad(x, ((0, NP - N), (0, 13)))
    batch3 = jnp.pad(batch, (0, NP - N), constant_values=G).astype(
        jnp.int32).reshape(NP // BN, 1, BN)

    p = params
    wr, wt, bias = {}, {}, {}
    for i in range(1, 6):
        s = p['bn%d_g' % i] / jnp.sqrt(1.0 + EPS)
        wr[i] = (p['conv%d_W_rel' % i] * s[:, None]).T.astype(jnp.bfloat16)
        wt[i] = (p['conv%d_W_root' % i] * s[:, None]).T.astype(jnp.bfloat16)
        bias[i] = (p['conv%d_b_rel' % i] * s + p['bn%d_b' % i])[None, :]
    wr[1] = jnp.pad(wr[1], ((0, 13), (0, 0)))
    wt[1] = jnp.pad(wt[1], ((0, 13), (0, 0)))

    p1 = _sc_pass(16, True, kj, kjbc)(
        lax.bitcast_convert_type(x_pad, jnp.int32), idx4)
    h1, invc = _layer1_call(p1, x_pad, wr[1], wt[1], bias[1])

    p2 = _sc_pass(32, False, kj, kjbc, bf16=True)(_pack_bf16(h1), idx4)
    (h2,) = _layer_call([p2], [h1], invc, [wr[2]], [wt[2]], bias[2], 64, 1)

    p3 = _sc_pass(64, False, kj, kjbc, bf16=True)(_pack_bf16(h2), idx4)
    (h3,) = _layer_call([p3], [h2], invc, [wr[3]], [wt[3]], bias[3], 128, 1)

    p4 = _sc_pass(128, False, kj, kjbc, bf16=True)(_pack_bf16(h3), idx4)
    h4a, h4b = _layer_call([p4], [h3], invc, [wr[4]], [wt[4]], bias[4],
                           256, 2)

    p5a = _sc_pass(128, False, kj, kjbc, bf16=True)(_pack_bf16(h4a), idx4)
    p5b = _sc_pass(128, False, kj, kjbc, bf16=True)(_pack_bf16(h4b), idx4)

    pooled = _layer5_pool_call(
        [p5a, p5b], [h4a, h4b], invc,
        [wr[5][:128], wr[5][128:]], [wt[5][:128], wt[5][128:]],
        bias[5], batch3)
    ws, bs = [], []
    for i in range(1, 4):
        wf, bf = _fold_bn(p['fc%d_W' % i], p['fc%d_b' % i],
                          p['bn_fc%d_g' % i], p['bn_fc%d_b' % i])
        ws.append(wf)
        bs.append(bf)
    ws.append(p['fc4_W'].T)
    bs.append(p['fc4_b'][None, :])
    return _mlp_call(pooled, ws, bs)


# speedup vs baseline: 1.0105x; 1.0105x over previous
import functools

import jax
import jax.numpy as jnp
from jax import lax
from jax.experimental import pallas as pl
from jax.experimental.pallas import tpu as pltpu
from jax.experimental.pallas import tpu_sc as plsc

N = 10000
G = 64
EPS = 1e-5
NP = 10240
NC, NS = 2, 16
NW = NC * NS
CH = 112
BN = 512
ROWS_PER_S = NP // NS
WBR = 80
WB_CH = ROWS_PER_S // WBR


def _sc_pass(D, with_cnt, KJ, kj_by_core, bf16=False, nh=1):
    mesh = plsc.VectorSubcoreMesh(
        core_axis_name="c", subcore_axis_name="s",
        num_cores=NC, num_subcores=NS)

    gw = D // 2 if bf16 else D
    scratch = [
        pltpu.VMEM((4, CH), jnp.int32),
        pltpu.VMEM((4, CH), jnp.int32),
        pltpu.VMEM((CH,), jnp.int32),
        pltpu.VMEM((CH,), jnp.int32),
        pltpu.VMEM((CH, gw), jnp.int32),
        pltpu.VMEM((CH, gw), jnp.int32),
        pltpu.VMEM((CH, D), jnp.float32),
        pltpu.VMEM((CH, D), jnp.float32),
        pltpu.VMEM_SHARED((NP, D), jnp.float32),
        pltpu.SemaphoreType.DMA,
        pltpu.SemaphoreType.DMA,
        pltpu.SemaphoreType.DMA,
        pltpu.SemaphoreType.DMA,
        pltpu.SemaphoreType.DMA,
        pltpu.SemaphoreType.DMA,
        pltpu.SemaphoreType.DMA,
        pltpu.SemaphoreType.DMA,
    ]

    def body(*args):
        h_hbms = args[:nh]
        (idx_hbm, out_hbm, ri0, ri1, sr0, sr1, bb0, bb1,
         buf0, buf1, acc, is0, is1, gs0a, gs0b, gs1a, gs1b,
         ss0, ss1) = args[nh:]
        gs0 = (gs0a, gs0b)
        gs1 = (gs1a, gs1b)
        c = lax.axis_index("c")
        s = lax.axis_index("s")
        w = s * NC + c

        zero16 = jnp.zeros((16,), jnp.float32)

        @pl.loop(0, CH)
        def _zero_buf(r):
            for k in range(D // 16):
                buf0[r, pl.ds(k * 16, 16)] = zero16

        lane = lax.iota(jnp.int32, 16)
        is_lane3 = lane == 3
        two_v = jnp.full((16,), 2, jnp.int32)
        three_v = jnp.full((16,), 3, jnp.int32)

        def run_half(h_hbm, ohalf):
            @pl.loop(0, WB_CH)
            def _zero_acc(t):
                pltpu.sync_copy(
                    buf0.at[pl.ds(0, WBR)],
                    acc.at[pl.ds(s * ROWS_PER_S + t * WBR, WBR)])

            plsc.subcore_barrier()
            _run_edges(h_hbm)
            plsc.subcore_barrier()

            @pl.loop(0, WB_CH)
            def _writeback(t):
                base = s * ROWS_PER_S + t * WBR
                pltpu.sync_copy(acc.at[pl.ds(base, WBR)],
                                buf0.at[pl.ds(0, WBR)])
                pltpu.sync_copy(buf0.at[pl.ds(0, WBR)],
                                ohalf.at[c, pl.ds(base, WBR)])

        def istart(j, ri, sem):
            pltpu.async_copy(idx_hbm.at[w, j], ri, sem)

        def iwait(ri, sem):
            pltpu.make_async_copy(idx_hbm.at[w, 0], ri, sem).wait()

        gh = CH // 2

        def gstart(h_hbm, ri, bb, sem):
            pltpu.async_copy(h_hbm.at[ri.at[0, pl.ds(0, gh)]],
                             bb.at[pl.ds(0, gh)], sem[0])
            pltpu.async_copy(h_hbm.at[ri.at[0, pl.ds(gh, gh)]],
                             bb.at[pl.ds(gh, gh)], sem[1])

        def gwait(h_hbm, bb, sem):
            pltpu.make_async_copy(h_hbm.at[ri0.at[0, pl.ds(0, gh)]],
                                  bb.at[pl.ds(0, gh)], sem[0]).wait()
            pltpu.make_async_copy(h_hbm.at[ri0.at[0, pl.ds(gh, gh)]],
                                  bb.at[pl.ds(gh, gh)], sem[1]).wait()

        def sstart(buf, sr, sem):
            pltpu.async_copy(buf, acc.at[sr], sem, add=True)

        def swait(buf, sem):
            pltpu.make_async_copy(buf, acc.at[sr0], sem).wait()

        hmask = jnp.int32(-65536)

        def scale(ri, bb, buf):
            @plsc.parallel_loop(0, CH, unroll=8)
            def _edge(e):
                ev = jnp.full((16,), e, jnp.int32)
                ews = plsc.bitcast(
                    plsc.load_gather(ri, [two_v, ev]), jnp.float32)
                if with_cnt:
                    vals = plsc.bitcast(
                        plsc.load_gather(ri, [three_v, ev]), jnp.float32)
                    row = plsc.bitcast(bb[e, pl.ds(0, 16)], jnp.float32)
                    buf[e, pl.ds(0, 16)] = jnp.where(
                        is_lane3, vals, row * ews)
                elif bf16:
                    for k in range(D // 32):
                        u = bb[e, pl.ds(k * 16, 16)]
                        lo = plsc.bitcast(lax.shift_left(u, 16),
                                          jnp.float32)
                        hi = plsc.bitcast(u & hmask, jnp.float32)
                        buf[e, pl.ds(k * 32, 16)] = lo * ews
                        buf[e, pl.ds(k * 32 + 16, 16)] = hi * ews
                else:
                    for k in range(D // 16):
                        buf[e, pl.ds(k * 16, 16)] = plsc.bitcast(
                            bb[e, pl.ds(k * 16, 16)], jnp.float32) * ews

        def dstcopy(ri, sr):
            for k in range(CH // 16):
                sr[pl.ds(k * 16, 16)] = ri[1, pl.ds(k * 16, 16)]

        def _run_edges(h_hbm):
            kjc = lax.select(c == 0, jnp.int32(kj_by_core[0]),
                             jnp.int32(kj_by_core[1]))
            pltpu.sync_copy(idx_hbm.at[w, 0], ri0)
            gstart(h_hbm, ri0, bb0, gs0)
            istart(1, ri1, is1)

            @pl.loop(0, kjc, step=2)
            def _pair(j):
                iwait(ri1, is1)

                @pl.when(j > 0)
                def _():
                    swait(buf1, ss1)
                gstart(h_hbm, ri1, bb1, gs1)
                gwait(h_hbm, bb0, gs0)

                @pl.when(j > 1)
                def _():
                    swait(buf0, ss0)
                scale(ri0, bb0, buf0)
                dstcopy(ri0, sr0)
                sstart(buf0, sr0, ss0)

                @pl.when(j + 2 < kjc)
                def _():
                    istart(j + 2, ri0, is0)

                gwait(h_hbm, bb1, gs1)
                scale(ri1, bb1, buf1)
                dstcopy(ri1, sr1)
                sstart(buf1, sr1, ss1)

                @pl.when(j + 2 < kjc)
                def _():
                    iwait(ri0, is0)
                    gstart(h_hbm, ri0, bb0, gs0)

                @pl.when(j + 3 < kjc)
                def _():
                    istart(j + 3, ri1, is1)

            swait(buf0, ss0)
            swait(buf1, ss1)

        if nh == 1:
            run_half(h_hbms[0], out_hbm)
        else:
            for t in range(nh):
                run_half(h_hbms[t], out_hbm.at[t])

    oshape = ((NC, NP, D) if nh == 1 else (nh, NC, NP, D))
    return pl.kernel(
        body,
        out_type=jax.ShapeDtypeStruct(oshape, jnp.float32),
        mesh=mesh,
        scratch_types=scratch,
        compiler_params=pltpu.CompilerParams(
            needs_layout_passes=False, use_tc_tiling_on_sc=False),
    )


def _layer1_call(p, x_pad, wr, wt, bias):
    def body(p_ref, x_ref, wr_ref, wt_ref, b_ref, out_ref, inv_ref):
        tot = p_ref[0] + p_ref[1]
        cnt = tot[:, 3:4]
        inv = 1.0 / jnp.maximum(cnt, 1.0)
        mean = tot * inv
        z = (jnp.dot(mean.astype(jnp.bfloat16), wr_ref[...],
                     preferred_element_type=jnp.float32)
             + jnp.dot(x_ref[...].astype(jnp.bfloat16), wt_ref[...],
                       preferred_element_type=jnp.float32)
             + b_ref[...])
        out_ref[...] = jnp.maximum(z, 0.0)
        inv_ref[...] = inv

    grid = NP // BN
    return pl.pallas_call(
        body,
        grid=(grid,),
        in_specs=[
            pl.BlockSpec((NC, BN, 16), lambda i: (0, i, 0)),
            pl.BlockSpec((BN, 16), lambda i: (i, 0)),
            pl.BlockSpec((16, 32), lambda i: (0, 0)),
            pl.BlockSpec((16, 32), lambda i: (0, 0)),
            pl.BlockSpec((1, 32), lambda i: (0, 0)),
        ],
        out_specs=[
            pl.BlockSpec((BN, 32), lambda i: (i, 0)),
            pl.BlockSpec((BN, 1), lambda i: (i, 0)),
        ],
        out_shape=[
            jax.ShapeDtypeStruct((NP, 32), jnp.float32),
            jax.ShapeDtypeStruct((NP, 1), jnp.float32),
        ],
    )(p, x_pad, wr, wt, bias)


def _layer_call(parts, hs, invc, wrs, wts, bias, dout, n_split):
    n_p, n_h = len(parts), len(hs)
    dps = [a.shape[2] for a in parts]
    dhs = [a.shape[1] for a in hs]
    wsp = dout // n_split

    def body(*refs):
        p_refs = refs[:n_p]
        h_refs = refs[n_p:n_p + n_h]
        inv_ref = refs[n_p + n_h]
        wr_refs = refs[n_p + n_h + 1: n_p + n_h + 1 + n_p]
        wt_refs = refs[n_p + n_h + 1 + n_p: n_p + n_h + 1 + n_p + n_h]
        b_ref = refs[n_p + n_h + 1 + n_p + n_h]
        out_refs = refs[n_p + n_h + 2 + n_p + n_h:]

        inv = inv_ref[...]
        z = b_ref[...]
        acc = None
        for pr, wr in zip(p_refs, wr_refs):
            mean = ((pr[0] + pr[1]) * inv).astype(jnp.bfloat16)
            t = jnp.dot(mean, wr[...], preferred_element_type=jnp.float32)
            acc = t if acc is None else acc + t
        for hr, wt in zip(h_refs, wt_refs):
            acc = acc + jnp.dot(hr[...].astype(jnp.bfloat16), wt[...],
                                preferred_element_type=jnp.float32)
        out = jnp.maximum(acc + z, 0.0)
        for k, o in enumerate(out_refs):
            o[...] = out[:, k * wsp:(k + 1) * wsp]

    grid = NP // BN
    in_specs = (
        [pl.BlockSpec((NC, BN, d), lambda i: (0, i, 0)) for d in dps]
        + [pl.BlockSpec((BN, d), lambda i: (i, 0)) for d in dhs]
        + [pl.BlockSpec((BN, 1), lambda i: (i, 0))]
        + [pl.BlockSpec((d, dout), lambda i: (0, 0)) for d in dps]
        + [pl.BlockSpec((d, dout), lambda i: (0, 0)) for d in dhs]
        + [pl.BlockSpec((1, dout), lambda i: (0, 0))]
    )
    out_specs = [pl.BlockSpec((BN, wsp), lambda i: (i, 0))
                 for _ in range(n_split)]
    out_shape = [jax.ShapeDtypeStruct((NP, wsp), jnp.float32)
                 for _ in range(n_split)]
    res = pl.pallas_call(
        body,
        grid=(grid,),
        in_specs=in_specs,
        out_specs=out_specs,
        out_shape=out_shape,
    )(*parts, *hs, invc, *wrs, *wts, bias)
    return res


def _layer5_pool_call(parts, hs, invc, wrs, wts, bias, batch3):
    nblk = NP // BN

    def body(pa, pb, ha, hb, inv_ref, wra, wrb, wta, wtb, b_ref, b3_ref,
             sum_ref, cnt_ref):
        i = pl.program_id(0)
        inv = inv_ref[...]
        acc = jnp.dot(((pa[0] + pa[1]) * inv).astype(jnp.bfloat16),
                      wra[...], preferred_element_type=jnp.float32)
        acc += jnp.dot(((pb[0] + pb[1]) * inv).astype(jnp.bfloat16),
                       wrb[...], preferred_element_type=jnp.float32)
        acc += jnp.dot(ha[...].astype(jnp.bfloat16), wta[...],
                       preferred_element_type=jnp.float32)
        acc += jnp.dot(hb[...].astype(jnp.bfloat16), wtb[...],
                       preferred_element_type=jnp.float32)
        out = jnp.maximum(acc + b_ref[...], 0.0)

        bv = b3_ref[0, 0, :]
        oh = (bv[:, None]
              == lax.broadcasted_iota(jnp.int32, (BN, G), 1)
              ).astype(jnp.float32)
        ps = lax.dot_general(oh, out, (((0,), (0,)), ((), ())),
                             preferred_element_type=jnp.float32)
        pc = jnp.sum(oh, axis=0)[:, None]

        @pl.when(i == 0)
        def _():
            sum_ref[...] = ps
            cnt_ref[...] = pc

        @pl.when(i != 0)
        def _():
            sum_ref[...] += ps
            cnt_ref[...] += pc

        @pl.when(i == nblk - 1)
        def _():
            sum_ref[...] = sum_ref[...] / jnp.maximum(cnt_ref[...], 1.0)

    return pl.pallas_call(
        body,
        grid=(nblk,),
        in_specs=[
            pl.BlockSpec((NC, BN, 128), lambda i: (0, i, 0)),
            pl.BlockSpec((NC, BN, 128), lambda i: (0, i, 0)),
            pl.BlockSpec((BN, 128), lambda i: (i, 0)),
            pl.BlockSpec((BN, 128), lambda i: (i, 0)),
            pl.BlockSpec((BN, 1), lambda i: (i, 0)),
            pl.BlockSpec((128, 512), lambda i: (0, 0)),
            pl.BlockSpec((128, 512), lambda i: (0, 0)),
            pl.BlockSpec((128, 512), lambda i: (0, 0)),
            pl.BlockSpec((128, 512), lambda i: (0, 0)),
            pl.BlockSpec((1, 512), lambda i: (0, 0)),
            pl.BlockSpec((1, 1, BN), lambda i: (i, 0, 0)),
        ],
        out_specs=[
            pl.BlockSpec((G, 512), lambda i: (0, 0)),
            pl.BlockSpec((G, 1), lambda i: (0, 0)),
        ],
        out_shape=[
            jax.ShapeDtypeStruct((G, 512), jnp.float32),
            jax.ShapeDtypeStruct((G, 1), jnp.float32),
        ],
    )(*parts, *hs, invc, *wrs, *wts, bias, batch3)[0]


def _pool_call(h5, batch3):
    nblk = NP // BN

    def body(h_ref, b_ref, sum_ref, cnt_ref):
        i = pl.program_id(0)
        bv = b_ref[0, 0, :]
        oh = (bv[:, None]
              == lax.broadcasted_iota(jnp.int32, (BN, G), 1)
              ).astype(jnp.float32)
        ps = lax.dot_general(oh, h_ref[...],
                             (((0,), (0,)), ((), ())),
                             preferred_element_type=jnp.float32)
        pc = jnp.sum(oh, axis=0)[:, None]

        @pl.when(i == 0)
        def _():
            sum_ref[...] = ps
            cnt_ref[...] = pc

        @pl.when(i != 0)
        def _():
            sum_ref[...] += ps
            cnt_ref[...] += pc

        @pl.when(i == nblk - 1)
        def _():
            sum_ref[...] = sum_ref[...] / jnp.maximum(cnt_ref[...], 1.0)

    return pl.pallas_call(
        body,
        grid=(nblk,),
        in_specs=[
            pl.BlockSpec((BN, 512), lambda i: (i, 0)),
            pl.BlockSpec((1, 1, BN), lambda i: (i, 0, 0)),
        ],
        out_specs=[
            pl.BlockSpec((G, 512), lambda i: (0, 0)),
            pl.BlockSpec((G, 1), lambda i: (0, 0)),
        ],
        out_shape=[
            jax.ShapeDtypeStruct((G, 512), jnp.float32),
            jax.ShapeDtypeStruct((G, 1), jnp.float32),
        ],
    )(h5, batch3)[0]


def _mlp_call(pooled, ws, bs):
    def body(x_ref, w1, b1, w2, b2, w3, b3, w4, b4, out_ref):
        h = x_ref[...]
        for wref, bref in ((w1, b1), (w2, b2), (w3, b3)):
            h = jnp.maximum(
                jnp.dot(h, wref[...], preferred_element_type=jnp.float32)
                + bref[...], 0.0)
        z = (jnp.dot(h, w4[...], preferred_element_type=jnp.float32)
             + b4[...])
        m = jnp.max(z, axis=1, keepdims=True)
        zz = z - m
        out_ref[...] = zz - jnp.log(
            jnp.sum(jnp.exp(zz), axis=1, keepdims=True))

    args = [pooled]
    for w, b in zip(ws, bs):
        args += [w, b]
    return pl.pallas_call(
        body,
        out_shape=jax.ShapeDtypeStruct((G, 2), jnp.float32),
    )(*args)


def _pack_bf16(h):
    npad, d = h.shape
    hb = h.reshape(npad, d // 32, 2, 16).swapaxes(-2, -1).astype(
        jnp.bfloat16)
    return lax.bitcast_convert_type(hb, jnp.int32).reshape(npad, d // 2)


def _fold_bn(w, b_lin, g, b_bn):
    s = g / jnp.sqrt(1.0 + EPS)
    wf = (w * s[:, None]).T
    bf = (b_lin * s + b_bn)[None, :]
    return wf, bf


def _slabify(a, kj0, kj1, kjm):
    a0 = a[:NS * kj0 * CH].reshape(NS, kj0, CH)
    a1 = a[NS * kj0 * CH:].reshape(NS, kj1, CH)
    a0 = jnp.pad(a0, ((0, 0), (0, kjm - kj0), (0, 0)))
    a1 = jnp.pad(a1, ((0, 0), (0, kjm - kj1), (0, 0)))
    return jnp.stack([a0, a1], axis=1).reshape(NW, kjm, CH)


def kernel(x, edge_index, edge_weight, edge_attr, batch, params):
    del edge_attr
    E = edge_index.shape[1]
    tch = 2 * ((E + 2 * NS * CH - 1) // (2 * NS * CH))
    kj0 = 2 * int(round(0.35 * tch))
    kj1 = tch - kj0
    kjm = max(kj0, kj1)
    e_cap = NS * CH * tch
    pe = e_cap - E

    src_f = jnp.pad(edge_index[0], (0, pe)).astype(jnp.int32)
    pad_dst = N + (jnp.arange(pe, dtype=jnp.int32) % (NP - N))
    dst_f = jnp.concatenate([edge_index[1].astype(jnp.int32), pad_dst])
    ew_f = jnp.pad(edge_weight, (0, pe))
    val_f = jnp.pad(jnp.ones((E,), jnp.float32), (0, pe))

    idx4 = jnp.stack(
        [_slabify(src_f, kj0, kj1, kjm),
         _slabify(dst_f, kj0, kj1, kjm),
         _slabify(lax.bitcast_convert_type(ew_f, jnp.int32),
                  kj0, kj1, kjm),
         _slabify(lax.bitcast_convert_type(val_f, jnp.int32),
                  kj0, kj1, kjm)], axis=2)
    kj = kjm
    kjbc = (kj0, kj1)

    x_pad = jnp.pad(x, ((0, NP - N), (0, 13)))
    batch3 = jnp.pad(batch, (0, NP - N), constant_values=G).astype(
        jnp.int32).reshape(NP // BN, 1, BN)

    p = params
    wr, wt, bias = {}, {}, {}
    for i in range(1, 6):
        s = p['bn%d_g' % i] / jnp.sqrt(1.0 + EPS)
        wr[i] = (p['conv%d_W_rel' % i] * s[:, None]).T.astype(jnp.bfloat16)
        wt[i] = (p['conv%d_W_root' % i] * s[:, None]).T.astype(jnp.bfloat16)
        bias[i] = (p['conv%d_b_rel' % i] * s + p['bn%d_b' % i])[None, :]
    wr[1] = jnp.pad(wr[1], ((0, 13), (0, 0)))
    wt[1] = jnp.pad(wt[1], ((0, 13), (0, 0)))

    p1 = _sc_pass(16, True, kj, kjbc)(
        lax.bitcast_convert_type(x_pad, jnp.int32), idx4)
    h1, invc = _layer1_call(p1, x_pad, wr[1], wt[1], bias[1])

    p2 = _sc_pass(32, False, kj, kjbc, bf16=True)(_pack_bf16(h1), idx4)
    (h2,) = _layer_call([p2], [h1], invc, [wr[2]], [wt[2]], bias[2], 64, 1)

    p3 = _sc_pass(64, False, kj, kjbc, bf16=True)(_pack_bf16(h2), idx4)
    (h3,) = _layer_call([p3], [h2], invc, [wr[3]], [wt[3]], bias[3], 128, 1)

    p4 = _sc_pass(128, False, kj, kjbc, bf16=True)(_pack_bf16(h3), idx4)
    h4a, h4b = _layer_call([p4], [h3], invc, [wr[4]], [wt[4]], bias[4],
                           256, 2)

    p5a = _sc_pass(128, False, kj, kjbc, bf16=True)(_pack_bf16(h4a), idx4)
    p5b = _sc_pass(128, False, kj, kjbc, bf16=True)(_pack_bf16(h4b), idx4)

    pooled = _layer5_pool_call(
        [p5a, p5b], [h4a, h4b], invc,
        [wr[5][:128], wr[5][128:]], [wt[5][:128], wt[5][128:]],
        bias[5], batch3)
    ws, bs = [], []
    for i in range(1, 4):
        wf, bf = _fold_bn(p['fc%d_W' % i], p['fc%d_b' % i],
                          p['bn_fc%d_g' % i], p['bn_fc%d_b' % i])
        ws.append(wf)
        bs.append(bf)
    ws.append(p['fc4_W'].T)
    bs.append(p['fc4_b'][None, :])
    return _mlp_call(pooled, ws, bs)


# speedup vs baseline: 1.0111x; 1.0007x over previous
import jax
import jax.numpy as jnp
from jax import lax
from jax.experimental import pallas as pl
from jax.experimental.pallas import tpu as pltpu
from jax.experimental.pallas import tpu_sc as plsc

N = 10000
G = 64
EPS = 1e-5
NP = 10240
NC, NS = 2, 16
NW = NC * NS
CH = 112
BN = 512
ROWS_PER_S = NP // NS
WBR = 80
WB_CH = ROWS_PER_S // WBR


def _sc_pass(D, with_cnt, KJ, kj_by_core, bf16=False, nh=1):
    mesh = plsc.VectorSubcoreMesh(
        core_axis_name="c", subcore_axis_name="s",
        num_cores=NC, num_subcores=NS)

    gw = D // 2 if bf16 else D
    scratch = [
        pltpu.VMEM((4, CH), jnp.int32),
        pltpu.VMEM((4, CH), jnp.int32),
        pltpu.VMEM((CH,), jnp.int32),
        pltpu.VMEM((CH,), jnp.int32),
        pltpu.VMEM((CH, gw), jnp.int32),
        pltpu.VMEM((CH, gw), jnp.int32),
        pltpu.VMEM((CH, D), jnp.float32),
        pltpu.VMEM((CH, D), jnp.float32),
        pltpu.VMEM_SHARED((NP, D), jnp.float32),
        pltpu.SemaphoreType.DMA,
        pltpu.SemaphoreType.DMA,
        pltpu.SemaphoreType.DMA,
        pltpu.SemaphoreType.DMA,
        pltpu.SemaphoreType.DMA,
        pltpu.SemaphoreType.DMA,
        pltpu.SemaphoreType.DMA,
        pltpu.SemaphoreType.DMA,
    ]

    def body(*args):
        h_hbms = args[:nh]
        (idx_hbm, out_hbm, ri0, ri1, sr0, sr1, bb0, bb1,
         buf0, buf1, acc, is0, is1, gs0a, gs0b, gs1a, gs1b,
         ss0, ss1) = args[nh:]
        gs0 = (gs0a, gs0b)
        gs1 = (gs1a, gs1b)
        c = lax.axis_index("c")
        s = lax.axis_index("s")
        w = s * NC + c

        zero16 = jnp.zeros((16,), jnp.float32)

        @pl.loop(0, CH)
        def _zero_buf(r):
            for k in range(D // 16):
                buf0[r, pl.ds(k * 16, 16)] = zero16

        lane = lax.iota(jnp.int32, 16)
        is_lane3 = lane == 3
        two_v = jnp.full((16,), 2, jnp.int32)
        three_v = jnp.full((16,), 3, jnp.int32)

        def run_half(h_hbm, ohalf):
            @pl.loop(0, WB_CH)
            def _zero_acc(t):
                pltpu.sync_copy(
                    buf0.at[pl.ds(0, WBR)],
                    acc.at[pl.ds(s * ROWS_PER_S + t * WBR, WBR)])

            plsc.subcore_barrier()
            _run_edges(h_hbm)
            plsc.subcore_barrier()

            @pl.loop(0, WB_CH)
            def _writeback(t):
                base = s * ROWS_PER_S + t * WBR
                pltpu.sync_copy(acc.at[pl.ds(base, WBR)],
                                buf0.at[pl.ds(0, WBR)])
                pltpu.sync_copy(buf0.at[pl.ds(0, WBR)],
                                ohalf.at[c, pl.ds(base, WBR)])

        def istart(j, ri, sem):
            pltpu.async_copy(idx_hbm.at[w, j], ri, sem)

        def iwait(ri, sem):
            pltpu.make_async_copy(idx_hbm.at[w, 0], ri, sem).wait()

        gh = CH // 2

        def gstart(h_hbm, ri, bb, sem):
            pltpu.async_copy(h_hbm.at[ri.at[0, pl.ds(0, gh)]],
                             bb.at[pl.ds(0, gh)], sem[0])
            pltpu.async_copy(h_hbm.at[ri.at[0, pl.ds(gh, gh)]],
                             bb.at[pl.ds(gh, gh)], sem[1])

        def gwait(h_hbm, bb, sem):
            pltpu.make_async_copy(h_hbm.at[ri0.at[0, pl.ds(0, gh)]],
                                  bb.at[pl.ds(0, gh)], sem[0]).wait()
            pltpu.make_async_copy(h_hbm.at[ri0.at[0, pl.ds(gh, gh)]],
                                  bb.at[pl.ds(gh, gh)], sem[1]).wait()

        def sstart(buf, sr, sem):
            pltpu.async_copy(buf, acc.at[sr], sem, add=True)

        def swait(buf, sem):
            pltpu.make_async_copy(buf, acc.at[sr0], sem).wait()

        hmask = jnp.int32(-65536)

        def scale(ri, bb, buf):
            @plsc.parallel_loop(0, CH, unroll=8)
            def _edge(e):
                ev = jnp.full((16,), e, jnp.int32)
                ews = plsc.bitcast(
                    plsc.load_gather(ri, [two_v, ev]), jnp.float32)
                if with_cnt:
                    vals = plsc.bitcast(
                        plsc.load_gather(ri, [three_v, ev]), jnp.float32)
                    row = plsc.bitcast(bb[e, pl.ds(0, 16)], jnp.float32)
                    buf[e, pl.ds(0, 16)] = jnp.where(
                        is_lane3, vals, row * ews)
                elif bf16:
                    for k in range(D // 32):
                        u = bb[e, pl.ds(k * 16, 16)]
                        lo = plsc.bitcast(lax.shift_left(u, 16),
                                          jnp.float32)
                        hi = plsc.bitcast(u & hmask, jnp.float32)
                        buf[e, pl.ds(k * 32, 16)] = lo * ews
                        buf[e, pl.ds(k * 32 + 16, 16)] = hi * ews
                else:
                    for k in range(D // 16):
                        buf[e, pl.ds(k * 16, 16)] = plsc.bitcast(
                            bb[e, pl.ds(k * 16, 16)], jnp.float32) * ews

        def dstcopy(ri, sr):
            for k in range(CH // 16):
                sr[pl.ds(k * 16, 16)] = ri[1, pl.ds(k * 16, 16)]

        def _run_edges(h_hbm):
            kjc = lax.select(c == 0, jnp.int32(kj_by_core[0]),
                             jnp.int32(kj_by_core[1]))
            pltpu.sync_copy(idx_hbm.at[w, 0], ri0)
            gstart(h_hbm, ri0, bb0, gs0)
            istart(1, ri1, is1)

            @pl.loop(0, kjc, step=2)
            def _pair(j):
                iwait(ri1, is1)

                @pl.when(j > 0)
                def _():
                    swait(buf1, ss1)
                gstart(h_hbm, ri1, bb1, gs1)
                gwait(h_hbm, bb0, gs0)

                @pl.when(j > 1)
                def _():
                    swait(buf0, ss0)
                scale(ri0, bb0, buf0)
                dstcopy(ri0, sr0)
                sstart(buf0, sr0, ss0)

                @pl.when(j + 2 < kjc)
                def _():
                    istart(j + 2, ri0, is0)

                gwait(h_hbm, bb1, gs1)
                scale(ri1, bb1, buf1)
                dstcopy(ri1, sr1)
                sstart(buf1, sr1, ss1)

                @pl.when(j + 2 < kjc)
                def _():
                    iwait(ri0, is0)
                    gstart(h_hbm, ri0, bb0, gs0)

                @pl.when(j + 3 < kjc)
                def _():
                    istart(j + 3, ri1, is1)

            swait(buf0, ss0)
            swait(buf1, ss1)

        if nh == 1:
            run_half(h_hbms[0], out_hbm)
        else:
            for t in range(nh):
                run_half(h_hbms[t], out_hbm.at[t])

    oshape = ((NC, NP, D) if nh == 1 else (nh, NC, NP, D))
    return pl.kernel(
        body,
        out_type=jax.ShapeDtypeStruct(oshape, jnp.float32),
        mesh=mesh,
        scratch_types=scratch,
        compiler_params=pltpu.CompilerParams(
            needs_layout_passes=False, use_tc_tiling_on_sc=False),
    )


def _layer1_call(p, x_pad, wr, wt, bias):
    def body(p_ref, x_ref, wr_ref, wt_ref, b_ref, out_ref, inv_ref):
        tot = p_ref[0] + p_ref[1]
        cnt = tot[:, 3:4]
        inv = 1.0 / jnp.maximum(cnt, 1.0)
        mean = tot * inv
        z = (jnp.dot(mean.astype(jnp.bfloat16), wr_ref[...],
                     preferred_element_type=jnp.float32)
             + jnp.dot(x_ref[...].astype(jnp.bfloat16), wt_ref[...],
                       preferred_element_type=jnp.float32)
             + b_ref[...])
        out_ref[...] = jnp.maximum(z, 0.0)
        inv_ref[...] = inv

    grid = NP // BN
    return pl.pallas_call(
        body,
        grid=(grid,),
        in_specs=[
            pl.BlockSpec((NC, BN, 16), lambda i: (0, i, 0)),
            pl.BlockSpec((BN, 16), lambda i: (i, 0)),
            pl.BlockSpec((16, 32), lambda i: (0, 0)),
            pl.BlockSpec((16, 32), lambda i: (0, 0)),
            pl.BlockSpec((1, 32), lambda i: (0, 0)),
        ],
        out_specs=[
            pl.BlockSpec((BN, 32), lambda i: (i, 0)),
            pl.BlockSpec((BN, 1), lambda i: (i, 0)),
        ],
        out_shape=[
            jax.ShapeDtypeStruct((NP, 32), jnp.float32),
            jax.ShapeDtypeStruct((NP, 1), jnp.float32),
        ],
    )(p, x_pad, wr, wt, bias)


def _layer_call(parts, hs, invc, wrs, wts, bias, dout, n_split):
    n_p, n_h = len(parts), len(hs)
    dps = [a.shape[2] for a in parts]
    dhs = [a.shape[1] for a in hs]
    wsp = dout // n_split

    def body(*refs):
        p_refs = refs[:n_p]
        h_refs = refs[n_p:n_p + n_h]
        inv_ref = refs[n_p + n_h]
        wr_refs = refs[n_p + n_h + 1: n_p + n_h + 1 + n_p]
        wt_refs = refs[n_p + n_h + 1 + n_p: n_p + n_h + 1 + n_p + n_h]
        b_ref = refs[n_p + n_h + 1 + n_p + n_h]
        out_refs = refs[n_p + n_h + 2 + n_p + n_h:]

        inv = inv_ref[...]
        z = b_ref[...]
        acc = None
        for pr, wr in zip(p_refs, wr_refs):
            mean = ((pr[0] + pr[1]) * inv).astype(jnp.bfloat16)
            t = jnp.dot(mean, wr[...], preferred_element_type=jnp.float32)
            acc = t if acc is None else acc + t
        for hr, wt in zip(h_refs, wt_refs):
            acc = acc + jnp.dot(hr[...].astype(jnp.bfloat16), wt[...],
                                preferred_element_type=jnp.float32)
        out = jnp.maximum(acc + z, 0.0)
        for k, o in enumerate(out_refs):
            o[...] = out[:, k * wsp:(k + 1) * wsp]

    grid = NP // BN
    in_specs = (
        [pl.BlockSpec((NC, BN, d), lambda i: (0, i, 0)) for d in dps]
        + [pl.BlockSpec((BN, d), lambda i: (i, 0)) for d in dhs]
        + [pl.BlockSpec((BN, 1), lambda i: (i, 0))]
        + [pl.BlockSpec((d, dout), lambda i: (0, 0)) for d in dps]
        + [pl.BlockSpec((d, dout), lambda i: (0, 0)) for d in dhs]
        + [pl.BlockSpec((1, dout), lambda i: (0, 0))]
    )
    out_specs = [pl.BlockSpec((BN, wsp), lambda i: (i, 0))
                 for _ in range(n_split)]
    out_shape = [jax.ShapeDtypeStruct((NP, wsp), jnp.float32)
                 for _ in range(n_split)]
    res = pl.pallas_call(
        body,
        grid=(grid,),
        in_specs=in_specs,
        out_specs=out_specs,
        out_shape=out_shape,
    )(*parts, *hs, invc, *wrs, *wts, bias)
    return res


def _layer5_pool_call(parts, hs, invc, wrs, wts, bias, batch3):
    nblk = NP // BN

    def body(pa, pb, ha, hb, inv_ref, wra, wrb, wta, wtb, b_ref, b3_ref,
             sum_ref, cnt_ref):
        i = pl.program_id(0)
        inv = inv_ref[...]
        acc = jnp.dot(((pa[0] + pa[1]) * inv).astype(jnp.bfloat16),
                      wra[...], preferred_element_type=jnp.float32)
        acc += jnp.dot(((pb[0] + pb[1]) * inv).astype(jnp.bfloat16),
                       wrb[...], preferred_element_type=jnp.float32)
        acc += jnp.dot(ha[...].astype(jnp.bfloat16), wta[...],
                       preferred_element_type=jnp.float32)
        acc += jnp.dot(hb[...].astype(jnp.bfloat16), wtb[...],
                       preferred_element_type=jnp.float32)
        out = jnp.maximum(acc + b_ref[...], 0.0)

        bv = b3_ref[0, 0, :]
        oh = (bv[:, None]
              == lax.broadcasted_iota(jnp.int32, (BN, G), 1)
              ).astype(jnp.float32)
        ps = lax.dot_general(oh, out, (((0,), (0,)), ((), ())),
                             preferred_element_type=jnp.float32)
        pc = jnp.sum(oh, axis=0)[:, None]

        @pl.when(i == 0)
        def _():
            sum_ref[...] = ps
            cnt_ref[...] = pc

        @pl.when(i != 0)
        def _():
            sum_ref[...] += ps
            cnt_ref[...] += pc

        @pl.when(i == nblk - 1)
        def _():
            sum_ref[...] = sum_ref[...] / jnp.maximum(cnt_ref[...], 1.0)

    return pl.pallas_call(
        body,
        grid=(nblk,),
        in_specs=[
            pl.BlockSpec((NC, BN, 128), lambda i: (0, i, 0)),
            pl.BlockSpec((NC, BN, 128), lambda i: (0, i, 0)),
            pl.BlockSpec((BN, 128), lambda i: (i, 0)),
            pl.BlockSpec((BN, 128), lambda i: (i, 0)),
            pl.BlockSpec((BN, 1), lambda i: (i, 0)),
            pl.BlockSpec((128, 512), lambda i: (0, 0)),
            pl.BlockSpec((128, 512), lambda i: (0, 0)),
            pl.BlockSpec((128, 512), lambda i: (0, 0)),
            pl.BlockSpec((128, 512), lambda i: (0, 0)),
            pl.BlockSpec((1, 512), lambda i: (0, 0)),
            pl.BlockSpec((1, 1, BN), lambda i: (i, 0, 0)),
        ],
        out_specs=[
            pl.BlockSpec((G, 512), lambda i: (0, 0)),
            pl.BlockSpec((G, 1), lambda i: (0, 0)),
        ],
        out_shape=[
            jax.ShapeDtypeStruct((G, 512), jnp.float32),
            jax.ShapeDtypeStruct((G, 1), jnp.float32),
        ],
    )(*parts, *hs, invc, *wrs, *wts, bias, batch3)[0]


def _mlp_call(pooled, ws, bs):
    def body(x_ref, w1, b1, w2, b2, w3, b3, w4, b4, out_ref):
        h = x_ref[...]
        for wref, bref in ((w1, b1), (w2, b2), (w3, b3)):
            h = jnp.maximum(
                jnp.dot(h, wref[...], preferred_element_type=jnp.float32)
                + bref[...], 0.0)
        z = (jnp.dot(h, w4[...], preferred_element_type=jnp.float32)
             + b4[...])
        m = jnp.max(z, axis=1, keepdims=True)
        zz = z - m
        out_ref[...] = zz - jnp.log(
            jnp.sum(jnp.exp(zz), axis=1, keepdims=True))

    args = [pooled]
    for w, b in zip(ws, bs):
        args += [w, b]
    return pl.pallas_call(
        body,
        out_shape=jax.ShapeDtypeStruct((G, 2), jnp.float32),
    )(*args)


def _pack_bf16(h):
    npad, d = h.shape
    hb = h.reshape(npad, d // 32, 2, 16).swapaxes(-2, -1).astype(
        jnp.bfloat16)
    return lax.bitcast_convert_type(hb, jnp.int32).reshape(npad, d // 2)


def _fold_bn(w, b_lin, g, b_bn):
    s = g / jnp.sqrt(1.0 + EPS)
    wf = (w * s[:, None]).T
    bf = (b_lin * s + b_bn)[None, :]
    return wf, bf


def _slabify(a, kj0, kj1, kjm):
    a0 = a[:NS * kj0 * CH].reshape(NS, kj0, CH)
    a1 = a[NS * kj0 * CH:].reshape(NS, kj1, CH)
    a0 = jnp.pad(a0, ((0, 0), (0, kjm - kj0), (0, 0)))
    a1 = jnp.pad(a1, ((0, 0), (0, kjm - kj1), (0, 0)))
    return jnp.stack([a0, a1], axis=1).reshape(NW, kjm, CH)


def kernel(x, edge_index, edge_weight, edge_attr, batch, params):
    del edge_attr
    E = edge_index.shape[1]
    tch = 2 * ((E + 2 * NS * CH - 1) // (2 * NS * CH))
    kj0 = 2 * int(round(0.35 * tch))
    kj1 = tch - kj0
    kjm = max(kj0, kj1)
    e_cap = NS * CH * tch
    pe = e_cap - E

    src_f = jnp.pad(edge_index[0], (0, pe)).astype(jnp.int32)
    pad_dst = N + (jnp.arange(pe, dtype=jnp.int32) % (NP - N))
    dst_f = jnp.concatenate([edge_index[1].astype(jnp.int32), pad_dst])
    ew_f = jnp.pad(edge_weight, (0, pe))
    val_f = jnp.pad(jnp.ones((E,), jnp.float32), (0, pe))

    idx4 = jnp.stack(
        [_slabify(src_f, kj0, kj1, kjm),
         _slabify(dst_f, kj0, kj1, kjm),
         _slabify(lax.bitcast_convert_type(ew_f, jnp.int32),
                  kj0, kj1, kjm),
         _slabify(lax.bitcast_convert_type(val_f, jnp.int32),
                  kj0, kj1, kjm)], axis=2)
    kj = kjm
    kjbc = (kj0, kj1)

    x_pad = jnp.pad(x, ((0, NP - N), (0, 13)))
    batch3 = jnp.pad(batch, (0, NP - N), constant_values=G).astype(
        jnp.int32).reshape(NP // BN, 1, BN)

    p = params
    wr, wt, bias = {}, {}, {}
    for i in range(1, 6):
        s = p['bn%d_g' % i] / jnp.sqrt(1.0 + EPS)
        wr[i] = (p['conv%d_W_rel' % i] * s[:, None]).T.astype(jnp.bfloat16)
        wt[i] = (p['conv%d_W_root' % i] * s[:, None]).T.astype(jnp.bfloat16)
        bias[i] = (p['conv%d_b_rel' % i] * s + p['bn%d_b' % i])[None, :]
    wr[1] = jnp.pad(wr[1], ((0, 13), (0, 0)))
    wt[1] = jnp.pad(wt[1], ((0, 13), (0, 0)))

    p1 = _sc_pass(16, True, kj, kjbc)(
        lax.bitcast_convert_type(x_pad, jnp.int32), idx4)
    h1, invc = _layer1_call(p1, x_pad, wr[1], wt[1], bias[1])

    p2 = _sc_pass(32, False, kj, kjbc, bf16=True)(_pack_bf16(h1), idx4)
    (h2,) = _layer_call([p2], [h1], invc, [wr[2]], [wt[2]], bias[2], 64, 1)

    p3 = _sc_pass(64, False, kj, kjbc, bf16=True)(_pack_bf16(h2), idx4)
    (h3,) = _layer_call([p3], [h2], invc, [wr[3]], [wt[3]], bias[3], 128, 1)

    p4 = _sc_pass(128, False, kj, kjbc, bf16=True)(_pack_bf16(h3), idx4)
    h4a, h4b = _layer_call([p4], [h3], invc, [wr[4]], [wt[4]], bias[4],
                           256, 2)

    p5a = _sc_pass(128, False, kj, kjbc, bf16=True)(_pack_bf16(h4a), idx4)
    p5b = _sc_pass(128, False, kj, kjbc, bf16=True)(_pack_bf16(h4b), idx4)

    pooled = _layer5_pool_call(
        [p5a, p5b], [h4a, h4b], invc,
        [wr[5][:128], wr[5][128:]], [wt[5][:128], wt[5][128:]],
        bias[5], batch3)
    ws, bs = [], []
    for i in range(1, 4):
        wf, bf = _fold_bn(p['fc%d_W' % i], p['fc%d_b' % i],
                          p['bn_fc%d_g' % i], p['bn_fc%d_b' % i])
        ws.append(wf)
        bs.append(bf)
    ws.append(p['fc4_W'].T)
    bs.append(p['fc4_b'][None, :])
    return _mlp_call(pooled, ws, bs)


# speedup vs baseline: 1.0473x; 1.0358x over previous
import jax
import jax.numpy as jnp
from jax import lax
from jax.experimental import pallas as pl
from jax.experimental.pallas import tpu as pltpu
from jax.experimental.pallas import tpu_sc as plsc

N = 10000
G = 64
EPS = 1e-5
NP = 10240
NC, NS = 2, 16
NW = NC * NS
CH = 112
BN = 512
ROWS_PER_S = NP // NS
WBR = 80
WB_CH = ROWS_PER_S // WBR


def _sc_pass(D, with_cnt, KJ, kj_by_core, bf16=False, nh=1):
    mesh = plsc.VectorSubcoreMesh(
        core_axis_name="c", subcore_axis_name="s",
        num_cores=NC, num_subcores=NS)

    gw = D // 2 if bf16 else D
    scratch = [
        pltpu.VMEM((4, CH), jnp.int32),
        pltpu.VMEM((4, CH), jnp.int32),
        pltpu.VMEM((CH,), jnp.int32),
        pltpu.VMEM((CH,), jnp.int32),
        pltpu.VMEM((CH, gw), jnp.int32),
        pltpu.VMEM((CH, gw), jnp.int32),
        pltpu.VMEM((CH, D), jnp.float32),
        pltpu.VMEM((CH, D), jnp.float32),
        pltpu.VMEM_SHARED((NP, D), jnp.float32),
        pltpu.SemaphoreType.DMA,
        pltpu.SemaphoreType.DMA,
        pltpu.SemaphoreType.DMA,
        pltpu.SemaphoreType.DMA,
        pltpu.SemaphoreType.DMA,
        pltpu.SemaphoreType.DMA,
        pltpu.SemaphoreType.DMA,
        pltpu.SemaphoreType.DMA,
    ]

    def body(*args):
        h_hbms = args[:nh]
        (idx_hbm, out_hbm, ri0, ri1, sr0, sr1, bb0, bb1,
         buf0, buf1, acc, is0, is1, gs0a, gs0b, gs1a, gs1b,
         ss0, ss1) = args[nh:]
        gs0 = (gs0a, gs0b)
        gs1 = (gs1a, gs1b)
        c = lax.axis_index("c")
        s = lax.axis_index("s")
        w = s * NC + c

        zero16 = jnp.zeros((16,), jnp.float32)

        @pl.loop(0, CH)
        def _zero_buf(r):
            for k in range(D // 16):
                buf0[r, pl.ds(k * 16, 16)] = zero16

        lane = lax.iota(jnp.int32, 16)
        is_lane3 = lane == 3
        two_v = jnp.full((16,), 2, jnp.int32)
        three_v = jnp.full((16,), 3, jnp.int32)

        def run_half(h_hbm, ohalf):
            @pl.loop(0, WB_CH)
            def _zero_acc(t):
                pltpu.sync_copy(
                    buf0.at[pl.ds(0, WBR)],
                    acc.at[pl.ds(s * ROWS_PER_S + t * WBR, WBR)])

            plsc.subcore_barrier()
            _run_edges(h_hbm)
            plsc.subcore_barrier()

            @pl.loop(0, WB_CH)
            def _writeback(t):
                base = s * ROWS_PER_S + t * WBR
                pltpu.sync_copy(acc.at[pl.ds(base, WBR)],
                                buf0.at[pl.ds(0, WBR)])
                pltpu.sync_copy(buf0.at[pl.ds(0, WBR)],
                                ohalf.at[c, pl.ds(base, WBR)])

        def istart(j, ri, sem):
            pltpu.async_copy(idx_hbm.at[w, j], ri, sem)

        def iwait(ri, sem):
            pltpu.make_async_copy(idx_hbm.at[w, 0], ri, sem).wait()

        gh = CH // 2

        def gstart(h_hbm, ri, bb, sem):
            pltpu.async_copy(h_hbm.at[ri.at[0, pl.ds(0, gh)]],
                             bb.at[pl.ds(0, gh)], sem[0])
            pltpu.async_copy(h_hbm.at[ri.at[0, pl.ds(gh, gh)]],
                             bb.at[pl.ds(gh, gh)], sem[1])

        def gwait(h_hbm, bb, sem):
            pltpu.make_async_copy(h_hbm.at[ri0.at[0, pl.ds(0, gh)]],
                                  bb.at[pl.ds(0, gh)], sem[0]).wait()
            pltpu.make_async_copy(h_hbm.at[ri0.at[0, pl.ds(gh, gh)]],
                                  bb.at[pl.ds(gh, gh)], sem[1]).wait()

        def sstart(buf, sr, sem):
            pltpu.async_copy(buf, acc.at[sr], sem, add=True)

        def swait(buf, sem):
            pltpu.make_async_copy(buf, acc.at[sr0], sem).wait()

        hmask = jnp.int32(-65536)

        def scale(ri, bb, buf):
            @plsc.parallel_loop(0, CH, unroll=8)
            def _edge(e):
                ev = jnp.full((16,), e, jnp.int32)
                ews = plsc.bitcast(
                    plsc.load_gather(ri, [two_v, ev]), jnp.float32)
                if with_cnt:
                    vals = plsc.bitcast(
                        plsc.load_gather(ri, [three_v, ev]), jnp.float32)
                    row = plsc.bitcast(bb[e, pl.ds(0, 16)], jnp.float32)
                    buf[e, pl.ds(0, 16)] = jnp.where(
                        is_lane3, vals, row * ews)
                elif bf16:
                    for k in range(D // 32):
                        u = bb[e, pl.ds(k * 16, 16)]
                        lo = plsc.bitcast(lax.shift_left(u, 16),
                                          jnp.float32)
                        hi = plsc.bitcast(u & hmask, jnp.float32)
                        buf[e, pl.ds(k * 32, 16)] = lo * ews
                        buf[e, pl.ds(k * 32 + 16, 16)] = hi * ews
                else:
                    for k in range(D // 16):
                        buf[e, pl.ds(k * 16, 16)] = plsc.bitcast(
                            bb[e, pl.ds(k * 16, 16)], jnp.float32) * ews

        def dstcopy(ri, sr):
            for k in range(CH // 16):
                sr[pl.ds(k * 16, 16)] = ri[1, pl.ds(k * 16, 16)]

        def _run_edges(h_hbm):
            kjc = lax.select(c == 0, jnp.int32(kj_by_core[0]),
                             jnp.int32(kj_by_core[1]))
            pltpu.sync_copy(idx_hbm.at[w, 0], ri0)
            gstart(h_hbm, ri0, bb0, gs0)
            istart(1, ri1, is1)

            @pl.loop(0, kjc, step=2)
            def _pair(j):
                iwait(ri1, is1)

                @pl.when(j > 0)
                def _():
                    swait(buf1, ss1)
                gstart(h_hbm, ri1, bb1, gs1)
                gwait(h_hbm, bb0, gs0)

                @pl.when(j > 1)
                def _():
                    swait(buf0, ss0)
                scale(ri0, bb0, buf0)
                dstcopy(ri0, sr0)
                sstart(buf0, sr0, ss0)

                @pl.when(j + 2 < kjc)
                def _():
                    istart(j + 2, ri0, is0)

                gwait(h_hbm, bb1, gs1)
                scale(ri1, bb1, buf1)
                dstcopy(ri1, sr1)
                sstart(buf1, sr1, ss1)

                @pl.when(j + 2 < kjc)
                def _():
                    iwait(ri0, is0)
                    gstart(h_hbm, ri0, bb0, gs0)

                @pl.when(j + 3 < kjc)
                def _():
                    istart(j + 3, ri1, is1)

            swait(buf0, ss0)
            swait(buf1, ss1)

        if nh == 1:
            run_half(h_hbms[0], out_hbm)
        else:
            for t in range(nh):
                run_half(h_hbms[t], out_hbm.at[t])

    oshape = ((NC, NP, D) if nh == 1 else (nh, NC, NP, D))
    return pl.kernel(
        body,
        out_type=jax.ShapeDtypeStruct(oshape, jnp.float32),
        mesh=mesh,
        scratch_types=scratch,
        compiler_params=pltpu.CompilerParams(
            needs_layout_passes=False, use_tc_tiling_on_sc=False),
    )


def _layer1_call(p, x_pad, wr, wt, bias):
    def body(p_ref, x_ref, wr_ref, wt_ref, b_ref, out_ref, inv_ref):
        tot = p_ref[0] + p_ref[1]
        cnt = tot[:, 3:4]
        inv = 1.0 / jnp.maximum(cnt, 1.0)
        mean = tot * inv
        z = (jnp.dot(mean.astype(jnp.bfloat16), wr_ref[...],
                     preferred_element_type=jnp.float32)
             + jnp.dot(x_ref[...].astype(jnp.bfloat16), wt_ref[...],
                       preferred_element_type=jnp.float32)
             + b_ref[...])
        out_ref[...] = jnp.maximum(z, 0.0)
        inv_ref[...] = inv

    grid = NP // BN
    return pl.pallas_call(
        body,
        grid=(grid,),
        in_specs=[
            pl.BlockSpec((NC, BN, 16), lambda i: (0, i, 0)),
            pl.BlockSpec((BN, 16), lambda i: (i, 0)),
            pl.BlockSpec((16, 32), lambda i: (0, 0)),
            pl.BlockSpec((16, 32), lambda i: (0, 0)),
            pl.BlockSpec((1, 32), lambda i: (0, 0)),
        ],
        out_specs=[
            pl.BlockSpec((BN, 32), lambda i: (i, 0)),
            pl.BlockSpec((BN, 1), lambda i: (i, 0)),
        ],
        out_shape=[
            jax.ShapeDtypeStruct((NP, 32), jnp.float32),
            jax.ShapeDtypeStruct((NP, 1), jnp.float32),
        ],
    )(p, x_pad, wr, wt, bias)


def _layer_call(parts, hs, invc, wrs, wts, bias, dout, n_split):
    n_p, n_h = len(parts), len(hs)
    dps = [a.shape[2] for a in parts]
    dhs = [a.shape[1] for a in hs]
    wsp = dout // n_split

    def body(*refs):
        p_refs = refs[:n_p]
        h_refs = refs[n_p:n_p + n_h]
        inv_ref = refs[n_p + n_h]
        wr_refs = refs[n_p + n_h + 1: n_p + n_h + 1 + n_p]
        wt_refs = refs[n_p + n_h + 1 + n_p: n_p + n_h + 1 + n_p + n_h]
        b_ref = refs[n_p + n_h + 1 + n_p + n_h]
        out_refs = refs[n_p + n_h + 2 + n_p + n_h:]

        inv = inv_ref[...]
        z = b_ref[...]
        acc = None
        for pr, wr in zip(p_refs, wr_refs):
            mean = ((pr[0] + pr[1]) * inv).astype(jnp.bfloat16)
            t = jnp.dot(mean, wr[...], preferred_element_type=jnp.float32)
            acc = t if acc is None else acc + t
        for hr, wt in zip(h_refs, wt_refs):
            acc = acc + jnp.dot(hr[...].astype(jnp.bfloat16), wt[...],
                                preferred_element_type=jnp.float32)
        out = jnp.maximum(acc + z, 0.0)
        for k, o in enumerate(out_refs):
            o[...] = out[:, k * wsp:(k + 1) * wsp]

    grid = NP // BN
    in_specs = (
        [pl.BlockSpec((NC, BN, d), lambda i: (0, i, 0)) for d in dps]
        + [pl.BlockSpec((BN, d), lambda i: (i, 0)) for d in dhs]
        + [pl.BlockSpec((BN, 1), lambda i: (i, 0))]
        + [pl.BlockSpec((d, dout), lambda i: (0, 0)) for d in dps]
        + [pl.BlockSpec((d, dout), lambda i: (0, 0)) for d in dhs]
        + [pl.BlockSpec((1, dout), lambda i: (0, 0))]
    )
    out_specs = [pl.BlockSpec((BN, wsp), lambda i: (i, 0))
                 for _ in range(n_split)]
    out_shape = [jax.ShapeDtypeStruct((NP, wsp), jnp.float32)
                 for _ in range(n_split)]
    res = pl.pallas_call(
        body,
        grid=(grid,),
        in_specs=in_specs,
        out_specs=out_specs,
        out_shape=out_shape,
    )(*parts, *hs, invc, *wrs, *wts, bias)
    return res


def _layer5_pool_call(parts, hs, invc, wrs, wts, bias, batch3):
    nblk = NP // BN

    def body(pa, pb, ha, hb, inv_ref, wra, wrb, wta, wtb, b_ref, b3_ref,
             sum_ref, cnt_ref):
        i = pl.program_id(0)
        inv = inv_ref[...]
        acc = jnp.dot(((pa[0] + pa[1]) * inv).astype(jnp.bfloat16),
                      wra[...], preferred_element_type=jnp.float32)
        acc += jnp.dot(((pb[0] + pb[1]) * inv).astype(jnp.bfloat16),
                       wrb[...], preferred_element_type=jnp.float32)
        acc += jnp.dot(ha[...].astype(jnp.bfloat16), wta[...],
                       preferred_element_type=jnp.float32)
        acc += jnp.dot(hb[...].astype(jnp.bfloat16), wtb[...],
                       preferred_element_type=jnp.float32)
        out = jnp.maximum(acc + b_ref[...], 0.0)

        bv = b3_ref[0, 0, :]
        oh = (bv[:, None]
              == lax.broadcasted_iota(jnp.int32, (BN, G), 1)
              ).astype(jnp.float32)
        ps = lax.dot_general(oh, out, (((0,), (0,)), ((), ())),
                             preferred_element_type=jnp.float32)
        pc = jnp.sum(oh, axis=0)[:, None]

        @pl.when(i == 0)
        def _():
            sum_ref[...] = ps
            cnt_ref[...] = pc

        @pl.when(i != 0)
        def _():
            sum_ref[...] += ps
            cnt_ref[...] += pc

        @pl.when(i == nblk - 1)
        def _():
            sum_ref[...] = sum_ref[...] / jnp.maximum(cnt_ref[...], 1.0)

    return pl.pallas_call(
        body,
        grid=(nblk,),
        in_specs=[
            pl.BlockSpec((NC, BN, 128), lambda i: (0, i, 0)),
            pl.BlockSpec((NC, BN, 128), lambda i: (0, i, 0)),
            pl.BlockSpec((BN, 128), lambda i: (i, 0)),
            pl.BlockSpec((BN, 128), lambda i: (i, 0)),
            pl.BlockSpec((BN, 1), lambda i: (i, 0)),
            pl.BlockSpec((128, 512), lambda i: (0, 0)),
            pl.BlockSpec((128, 512), lambda i: (0, 0)),
            pl.BlockSpec((128, 512), lambda i: (0, 0)),
            pl.BlockSpec((128, 512), lambda i: (0, 0)),
            pl.BlockSpec((1, 512), lambda i: (0, 0)),
            pl.BlockSpec((1, 1, BN), lambda i: (i, 0, 0)),
        ],
        out_specs=[
            pl.BlockSpec((G, 512), lambda i: (0, 0)),
            pl.BlockSpec((G, 1), lambda i: (0, 0)),
        ],
        out_shape=[
            jax.ShapeDtypeStruct((G, 512), jnp.float32),
            jax.ShapeDtypeStruct((G, 1), jnp.float32),
        ],
    )(*parts, *hs, invc, *wrs, *wts, bias, batch3)[0]


def _mlp_call(pooled, ws, bs):
    def body(x_ref, w1, b1, w2, b2, w3, b3, w4, b4, out_ref):
        h = x_ref[...]
        for wref, bref in ((w1, b1), (w2, b2), (w3, b3)):
            h = jnp.maximum(
                jnp.dot(h, wref[...], preferred_element_type=jnp.float32)
                + bref[...], 0.0)
        z = (jnp.dot(h, w4[...], preferred_element_type=jnp.float32)
             + b4[...])
        m = jnp.max(z, axis=1, keepdims=True)
        zz = z - m
        out_ref[...] = zz - jnp.log(
            jnp.sum(jnp.exp(zz), axis=1, keepdims=True))

    args = [pooled]
    for w, b in zip(ws, bs):
        args += [w, b]
    return pl.pallas_call(
        body,
        out_shape=jax.ShapeDtypeStruct((G, 2), jnp.float32),
    )(*args)


def _pack_bf16(h):
    npad, d = h.shape
    hb = h.reshape(npad, d // 32, 2, 16).swapaxes(-2, -1).astype(
        jnp.bfloat16)
    return lax.bitcast_convert_type(hb, jnp.int32).reshape(npad, d // 2)


def _fold_bn(w, b_lin, g, b_bn):
    s = g / jnp.sqrt(1.0 + EPS)
    wf = (w * s[:, None]).T
    bf = (b_lin * s + b_bn)[None, :]
    return wf, bf


def _slabify(a, kj0, kj1, kjm):
    a0 = a[:NS * kj0 * CH].reshape(NS, kj0, CH)
    a1 = a[NS * kj0 * CH:].reshape(NS, kj1, CH)
    a0 = jnp.pad(a0, ((0, 0), (0, kjm - kj0), (0, 0)))
    a1 = jnp.pad(a1, ((0, 0), (0, kjm - kj1), (0, 0)))
    return jnp.stack([a0, a1], axis=1).reshape(NW, kjm, CH)


def kernel(x, edge_index, edge_weight, edge_attr, batch, params):
    del edge_attr
    E = edge_index.shape[1]
    tch = 2 * ((E + 2 * NS * CH - 1) // (2 * NS * CH))
    kj0 = 2 * int(round(0.25 * tch))
    kj1 = tch - kj0
    kjm = max(kj0, kj1)
    e_cap = NS * CH * tch
    pe = e_cap - E

    src_f = jnp.pad(edge_index[0], (0, pe)).astype(jnp.int32)
    pad_dst = N + (jnp.arange(pe, dtype=jnp.int32) % (NP - N))
    dst_f = jnp.concatenate([edge_index[1].astype(jnp.int32), pad_dst])
    ew_f = jnp.pad(edge_weight, (0, pe))
    val_f = jnp.pad(jnp.ones((E,), jnp.float32), (0, pe))

    idx4 = jnp.stack(
        [_slabify(src_f, kj0, kj1, kjm),
         _slabify(dst_f, kj0, kj1, kjm),
         _slabify(lax.bitcast_convert_type(ew_f, jnp.int32),
                  kj0, kj1, kjm),
         _slabify(lax.bitcast_convert_type(val_f, jnp.int32),
                  kj0, kj1, kjm)], axis=2)
    kj = kjm
    kjbc = (kj0, kj1)

    x_pad = jnp.pad(x, ((0, NP - N), (0, 13)))
    batch3 = jnp.pad(batch, (0, NP - N), constant_values=G).astype(
        jnp.int32).reshape(NP // BN, 1, BN)

    p = params
    wr, wt, bias = {}, {}, {}
    for i in range(1, 6):
        s = p['bn%d_g' % i] / jnp.sqrt(1.0 + EPS)
        wr[i] = (p['conv%d_W_rel' % i] * s[:, None]).T.astype(jnp.bfloat16)
        wt[i] = (p['conv%d_W_root' % i] * s[:, None]).T.astype(jnp.bfloat16)
        bias[i] = (p['conv%d_b_rel' % i] * s + p['bn%d_b' % i])[None, :]
    wr[1] = jnp.pad(wr[1], ((0, 13), (0, 0)))
    wt[1] = jnp.pad(wt[1], ((0, 13), (0, 0)))

    p1 = _sc_pass(16, True, kj, kjbc)(
        lax.bitcast_convert_type(x_pad, jnp.int32), idx4)
    h1, invc = _layer1_call(p1, x_pad, wr[1], wt[1], bias[1])

    p2 = _sc_pass(32, False, kj, kjbc, bf16=True)(_pack_bf16(h1), idx4)
    (h2,) = _layer_call([p2], [h1], invc, [wr[2]], [wt[2]], bias[2], 64, 1)

    p3 = _sc_pass(64, False, kj, kjbc, bf16=True)(_pack_bf16(h2), idx4)
    (h3,) = _layer_call([p3], [h2], invc, [wr[3]], [wt[3]], bias[3], 128, 1)

    p4 = _sc_pass(128, False, kj, kjbc, bf16=True)(_pack_bf16(h3), idx4)
    h4a, h4b = _layer_call([p4], [h3], invc, [wr[4]], [wt[4]], bias[4],
                           256, 2)

    p5a = _sc_pass(128, False, kj, kjbc, bf16=True)(_pack_bf16(h4a), idx4)
    p5b = _sc_pass(128, False, kj, kjbc, bf16=True)(_pack_bf16(h4b), idx4)

    pooled = _layer5_pool_call(
        [p5a, p5b], [h4a, h4b], invc,
        [wr[5][:128], wr[5][128:]], [wt[5][:128], wt[5][128:]],
        bias[5], batch3)
    ws, bs = [], []
    for i in range(1, 4):
        wf, bf = _fold_bn(p['fc%d_W' % i], p['fc%d_b' % i],
                          p['bn_fc%d_g' % i], p['bn_fc%d_b' % i])
        ws.append(wf)
        bs.append(bf)
    ws.append(p['fc4_W'].T)
    bs.append(p['fc4_b'][None, :])
    return _mlp_call(pooled, ws, bs)
